# baseline (speedup 1.0000x reference)
import jax
import jax.numpy as jnp
from jax import lax
from jax.experimental import pallas as pl
from jax.experimental.pallas import tpu as pltpu
from jax.experimental.pallas import tpu_sc as plsc

_B = 16384
_D = 64
_NC = 2
_NS = 16
_NW = _NC * _NS
_PER_W = _B // _NW
_C = 128
_NCHUNK = _PER_W // _C
_L = 16


def _rsqrt(x):
    i = plsc.bitcast(x, jnp.int32)
    i = jnp.int32(0x5F3759DF) - lax.shift_right_arithmetic(i, 1)
    y = plsc.bitcast(i, jnp.float32)
    for _ in range(3):
        y = y * (1.5 - 0.5 * x * y * y)
    return y


def _body(h_hbm, r_hbm, t_hbm, ent_hbm, rel_hbm, nrm_hbm, out_hbm,
          hidx, tidx, ridx, hrow, trow, rrow, nrow, obuf, sem):
    wid = lax.axis_index("s") * _NC + lax.axis_index("c")
    for c in range(_NCHUNK):
        base = wid * _PER_W + c * _C
        pltpu.sync_copy(h_hbm.at[pl.ds(base, _C)], hidx)
        pltpu.sync_copy(t_hbm.at[pl.ds(base, _C)], tidx)
        pltpu.sync_copy(r_hbm.at[pl.ds(base, _C)], ridx)
        cp_h = pltpu.async_copy(ent_hbm.at[hidx], hrow, sem)
        cp_t = pltpu.async_copy(ent_hbm.at[tidx], trow, sem)
        cp_r = pltpu.async_copy(rel_hbm.at[ridx], rrow, sem)
        cp_n = pltpu.async_copy(nrm_hbm.at[ridx], nrow, sem)
        cp_h.wait()
        cp_t.wait()
        cp_r.wait()
        cp_n.wait()

        def group(g, carry):
            rowv = lax.iota(jnp.int32, _L) + g * _L
            zero = jnp.zeros((_L,), jnp.float32)
            nn, ne, un, uu = zero, zero, zero, zero
            for j in range(_D):
                colv = jnp.full((_L,), j, jnp.int32)
                hj = plsc.load_gather(hrow, [rowv, colv])
                tj = plsc.load_gather(trow, [rowv, colv])
                nj = plsc.load_gather(nrow, [rowv, colv])
                rj = plsc.load_gather(rrow, [rowv, colv])
                e = hj - tj
                u = e + rj
                nn = nn + nj * nj
                ne = ne + nj * e
                un = un + nj * u
                uu = uu + u * u
            s = nn * _rsqrt(nn)
            a = 1.0 / (s + 1e-12)
            coef = ne * a * a
            dd = uu - 2.0 * coef * un + coef * coef * nn
            dd = jnp.maximum(dd, 0.0)
            obuf[pl.ds(g * _L, _L)] = dd * _rsqrt(dd)
            return carry

        lax.fori_loop(0, _C // _L, group, 0)
        pltpu.sync_copy(obuf, out_hbm.at[pl.ds(base, _C)])


@jax.jit
def _transh_sc(h, r, t, ent, rel, nrm):
    mesh = plsc.VectorSubcoreMesh(core_axis_name="c", subcore_axis_name="s")
    return pl.kernel(
        _body,
        out_type=jax.ShapeDtypeStruct((_B,), jnp.float32),
        mesh=mesh,
        compiler_params=pltpu.CompilerParams(
            needs_layout_passes=False, use_tc_tiling_on_sc=False),
        scratch_types=[
            pltpu.VMEM((_C,), jnp.int32),
            pltpu.VMEM((_C,), jnp.int32),
            pltpu.VMEM((_C,), jnp.int32),
            pltpu.VMEM((_C, _D), jnp.float32),
            pltpu.VMEM((_C, _D), jnp.float32),
            pltpu.VMEM((_C, _D), jnp.float32),
            pltpu.VMEM((_C, _D), jnp.float32),
            pltpu.VMEM((_C,), jnp.float32),
            pltpu.SemaphoreType.DMA,
        ],
    )(h, r, t, ent, rel, nrm)


def kernel(h, r, t, emb_entity, emb_relation, emb_normal_vec):
    h = h.astype(jnp.int32)
    r = r.astype(jnp.int32)
    t = t.astype(jnp.int32)
    return _transh_sc(h, r, t, emb_entity, emb_relation, emb_normal_vec)
